# baseline (speedup 1.0000x reference)
import jax
import jax.numpy as jnp
from jax.experimental import pallas as pl
from jax.experimental.pallas import tpu as pltpu

_ROWS = 2 * 8192
_COLS = 2048
_BLK = 1024
_NBUF = 3
_NSTEP = _ROWS // _BLK


def _start(x_hbm, y_hbm, xb, yb, sems, j, s):
    pltpu.make_async_copy(
        x_hbm.at[pl.ds(j * _BLK, _BLK)], xb.at[s], sems.at[s, 0]).start()
    pltpu.make_async_copy(
        y_hbm.at[pl.ds(j * _BLK, _BLK)], yb.at[s], sems.at[s, 1]).start()


def _wait(x_hbm, y_hbm, xb, yb, sems, s):
    pltpu.make_async_copy(
        x_hbm.at[pl.ds(0, _BLK)], xb.at[s], sems.at[s, 0]).wait()
    pltpu.make_async_copy(
        y_hbm.at[pl.ds(0, _BLK)], yb.at[s], sems.at[s, 1]).wait()


def _reduce_kernel(x_hbm, y_hbm, o_ref, xb, yb, sems, acc_ref):
    i = pl.program_id(0)

    @pl.when(i == 0)
    def _init():
        acc_ref[0] = 0.0
        acc_ref[1] = 0.0
        for k in range(_NBUF - 1):
            _start(x_hbm, y_hbm, xb, yb, sems, k, k)

    j = i + _NBUF - 1

    @pl.when(j < _NSTEP)
    def _prefetch():
        _start(x_hbm, y_hbm, xb, yb, sems, j, j % _NBUF)

    s = i % _NBUF
    _wait(x_hbm, y_hbm, xb, yb, sems, s)
    x = xb[s]
    y = yb[s]
    d = x - y
    acc_ref[0] += jnp.sum(d * d)
    acc_ref[1] += jnp.sum(y * y)

    @pl.when(i == _NSTEP - 1)
    def _fin():
        o_ref[0] = acc_ref[0] / acc_ref[1]


def kernel(x, y):
    xf = x.reshape(_ROWS, _COLS)
    yf = y.reshape(_ROWS, _COLS)
    out = pl.pallas_call(
        _reduce_kernel,
        grid=(_NSTEP,),
        in_specs=[
            pl.BlockSpec(memory_space=pl.ANY),
            pl.BlockSpec(memory_space=pl.ANY),
        ],
        out_specs=pl.BlockSpec(memory_space=pltpu.SMEM),
        out_shape=jax.ShapeDtypeStruct((1,), jnp.float32),
        scratch_shapes=[
            pltpu.VMEM((_NBUF, _BLK, _COLS), jnp.float32),
            pltpu.VMEM((_NBUF, _BLK, _COLS), jnp.float32),
            pltpu.SemaphoreType.DMA((_NBUF, 2)),
            pltpu.SMEM((2,), jnp.float32),
        ],
        compiler_params=pltpu.CompilerParams(
            dimension_semantics=("arbitrary",)),
    )(xf, yf)
    return out[0]


# speedup vs baseline: 1.0221x; 1.0221x over previous
import jax
import jax.numpy as jnp
from jax.experimental import pallas as pl
from jax.experimental.pallas import tpu as pltpu

_ROWS = 2 * 8192
_COLS = 2048
_BLK = 128
_NBUF = 16
_NSTEP = _ROWS // _BLK


def _start(x_hbm, y_hbm, xb, yb, sems, j, s):
    pltpu.make_async_copy(
        x_hbm.at[pl.ds(j * _BLK, _BLK)], xb.at[s], sems.at[s, 0]).start()
    pltpu.make_async_copy(
        y_hbm.at[pl.ds(j * _BLK, _BLK)], yb.at[s], sems.at[s, 1]).start()


def _wait(x_hbm, y_hbm, xb, yb, sems, s):
    pltpu.make_async_copy(
        x_hbm.at[pl.ds(0, _BLK)], xb.at[s], sems.at[s, 0]).wait()
    pltpu.make_async_copy(
        y_hbm.at[pl.ds(0, _BLK)], yb.at[s], sems.at[s, 1]).wait()


def _reduce_kernel(x_hbm, y_hbm, o_ref, xb, yb, sems, acc_ref):
    i = pl.program_id(0)

    @pl.when(i == 0)
    def _init():
        acc_ref[0] = 0.0
        acc_ref[1] = 0.0
        for k in range(_NBUF - 1):
            _start(x_hbm, y_hbm, xb, yb, sems, k, k)

    j = i + _NBUF - 1

    @pl.when(j < _NSTEP)
    def _prefetch():
        _start(x_hbm, y_hbm, xb, yb, sems, j, j % _NBUF)

    s = i % _NBUF
    _wait(x_hbm, y_hbm, xb, yb, sems, s)
    x = xb[s]
    y = yb[s]
    d = x - y
    acc_ref[0] += jnp.sum(d * d)
    acc_ref[1] += jnp.sum(y * y)

    @pl.when(i == _NSTEP - 1)
    def _fin():
        o_ref[0] = acc_ref[0] / acc_ref[1]


def kernel(x, y):
    xf = x.reshape(_ROWS, _COLS)
    yf = y.reshape(_ROWS, _COLS)
    out = pl.pallas_call(
        _reduce_kernel,
        grid=(_NSTEP,),
        in_specs=[
            pl.BlockSpec(memory_space=pl.ANY),
            pl.BlockSpec(memory_space=pl.ANY),
        ],
        out_specs=pl.BlockSpec(memory_space=pltpu.SMEM),
        out_shape=jax.ShapeDtypeStruct((1,), jnp.float32),
        scratch_shapes=[
            pltpu.VMEM((_NBUF, _BLK, _COLS), jnp.float32),
            pltpu.VMEM((_NBUF, _BLK, _COLS), jnp.float32),
            pltpu.SemaphoreType.DMA((_NBUF, 2)),
            pltpu.SMEM((2,), jnp.float32),
        ],
        compiler_params=pltpu.CompilerParams(
            dimension_semantics=("arbitrary",)),
    )(xf, yf)
    return out[0]
